# baseline (speedup 1.0000x reference)
import jax
import jax.numpy as jnp
from jax import lax
from jax.experimental import pallas as pl
from jax.experimental.pallas import tpu as pltpu
from jax.experimental.pallas import tpu_sc as plsc
from jax._src.pallas import mpmd as _mpmd

NROWS = 1_000_000
NFEAT = 16
NIDX = 16384
NC = 2
NS = 16
L = 16
NW = NC * NS
NCHUNKS = NIDX // L


def _body(x_hbm, idx_hbm, src_hbm, x_io,
          idx_v, pos_v, rowscr, rowidx, svidx, stage_x, stage_s, semA, semB):
    wid = (lax.axis_index("s") * NC + lax.axis_index("c")).astype(jnp.int32)
    iota = lax.iota(jnp.int32, L)

    pltpu.sync_copy(idx_hbm, idx_v)

    def scan1(i, k):
        r = idx_v[pl.ds(i * L, L)]
        m = (r & (NW - 1)) == wid
        plsc.store_compressed(pos_v.at[pl.ds(k, L)], i * L + iota, mask=m)
        return k + jnp.sum(m.astype(jnp.int32))

    k = lax.fori_loop(0, NCHUNKS, scan1, jnp.int32(0))
    nb = (k + (L - 1)) // L

    def batch_body(j, _):
        valid = (j * L + iota) < k
        pos = jnp.where(valid, pos_v[pl.ds(j * L, L)], 0)
        r = plsc.load_gather(idx_v, [pos])
        rowscr[...] = r
        rank, _last = plsc.scan_count(r, mask=valid)
        nr = jnp.max(jnp.where(valid, rank, 0))

        def round_body(rd, _):
            active = valid & (rank == rd)
            a = jnp.min(jnp.where(active, iota, L - 1))
            avec = jnp.full((L,), 0, jnp.int32) + a
            arow = plsc.load_gather(rowscr, [avec])
            rowidx[...] = jnp.where(active, r, arow)
            svidx[...] = jnp.where(active, pos, 0)
            g1 = pltpu.async_copy(x_io.at[rowidx], stage_x, semA)
            g2 = pltpu.async_copy(src_hbm.at[svidx], stage_s, semB)
            g1.wait()
            g2.wait()
            inact = jnp.logical_not(active)
            for f in range(NFEAT):
                fvec = jnp.full((L,), f, jnp.int32)
                colS = plsc.load_gather(stage_s, [iota, fvec])
                plsc.addupdate_scatter(stage_x, [iota, fvec], colS, mask=active)
                vf = plsc.load_gather(stage_x, [avec, fvec])
                plsc.store_scatter(stage_x, [iota, fvec], vf, mask=inact)
            pltpu.async_copy(stage_x, x_io.at[rowidx], semA).wait()
            return 0

        lax.fori_loop(0, nr + 1, round_body, 0)
        return 0

    lax.fori_loop(0, nb, batch_body, 0)


def _make_kernel():
    mesh = plsc.VectorSubcoreMesh(
        core_axis_name="c", subcore_axis_name="s", num_cores=NC, num_subcores=NS)
    return _mpmd._mpmd_map(
        [(mesh, _body)],
        jax.ShapeDtypeStruct((NROWS, NFEAT), jnp.float32),
        input_output_aliases={0: 0},
        compiler_params=pltpu.CompilerParams(
            needs_layout_passes=False, use_tc_tiling_on_sc=False),
        scratch_types=[
            pltpu.VMEM((NIDX,), jnp.int32),
            pltpu.VMEM((NIDX,), jnp.int32),
            pltpu.VMEM((L,), jnp.int32),
            pltpu.VMEM((L,), jnp.int32),
            pltpu.VMEM((L,), jnp.int32),
            pltpu.VMEM((L, NFEAT), jnp.float32),
            pltpu.VMEM((L, NFEAT), jnp.float32),
            pltpu.SemaphoreType.DMA,
            pltpu.SemaphoreType.DMA,
        ],
    )


def kernel(x, idx, src):
    idx32 = idx.astype(jnp.int32)
    return _make_kernel()(x, idx32, src)
